# Initial kernel scaffold; baseline (speedup 1.0000x reference)
import jax
import jax.numpy as jnp
from jax.experimental import pallas as pl


def kernel(enc, src_lens, tgt_lens):
    raise NotImplementedError("write your pallas kernel here")


# speedup vs baseline: 1.3418x; 1.3418x over previous
import functools

import jax
import jax.numpy as jnp
from jax import lax
from jax.experimental import pallas as pl
from jax.experimental.pallas import tpu as pltpu
from jax.experimental.pallas import tpu_sc as plsc

B, S, T, D = 16, 4096, 4096, 1024
L = 16
NC, NS = 2, 16
NW = NC * NS
ROWS_PER_W = (B * T) // NW
K = 32
NCHUNK = ROWS_PER_W // K
VPC = K // L


def _sc_body(enc_hbm, steps_hbm, tgt_hbm, out_hbm, mask_hbm,
             steps_v, tgt_v, idx_a, idx_b, rows_a, rows_b, mask_v,
             sem_a, sem_b):
    wid = lax.axis_index("s") * NC + lax.axis_index("c")
    b = wid // 2
    t0 = (wid % 2) * ROWS_PER_W
    row0 = wid * ROWS_PER_W

    pltpu.sync_copy(steps_hbm.at[wid], steps_v)
    pltpu.sync_copy(tgt_hbm.at[wid], tgt_v)
    steps = steps_v[...]
    tgt = tgt_v[...]
    base_flat = jnp.full((L,), b * S, jnp.int32)

    def compute_chunk(c, idx_ref):
        for j in range(VPC):
            t_i = t0 + c * K + j * L + lax.iota(jnp.int32, 16)
            x = steps * t_i.astype(jnp.float32)
            f = x.astype(jnp.int32)
            fr = x - f.astype(jnp.float32)
            half = jnp.float32(0.5)
            odd = (f & 1) == 1
            inc = jnp.where((fr > half) | ((fr == half) & odd), 1, 0)
            r = jnp.minimum(jnp.maximum(f + inc, 0), S - 1)
            m = t_i < tgt
            idx_ref[pl.ds(j * L, L)] = base_flat + jnp.where(m, r, 1)
            mask_v[pl.ds(c * K + j * L, L)] = jnp.where(
                m, jnp.float32(1.0), jnp.float32(0.0))

    def start_gather(idx_ref, rows_ref, sem):
        pltpu.make_async_copy(enc_hbm.at[idx_ref], rows_ref, sem).start()

    def drain(c, idx_ref, rows_ref, sem):
        pltpu.make_async_copy(enc_hbm.at[idx_ref], rows_ref, sem).wait()
        pltpu.sync_copy(rows_ref, out_hbm.at[pl.ds(row0 + c * K, K)])

    compute_chunk(0, idx_a)
    start_gather(idx_a, rows_a, sem_a)

    def loop_body(k, carry):
        c0 = 2 * k
        compute_chunk(c0 + 1, idx_b)
        start_gather(idx_b, rows_b, sem_b)
        drain(c0, idx_a, rows_a, sem_a)
        compute_chunk(c0 + 2, idx_a)
        start_gather(idx_a, rows_a, sem_a)
        drain(c0 + 1, idx_b, rows_b, sem_b)
        return carry

    lax.fori_loop(0, NCHUNK // 2 - 1, loop_body, 0)

    compute_chunk(NCHUNK - 1, idx_b)
    start_gather(idx_b, rows_b, sem_b)
    drain(NCHUNK - 2, idx_a, rows_a, sem_a)
    drain(NCHUNK - 1, idx_b, rows_b, sem_b)

    pltpu.sync_copy(mask_v, mask_hbm.at[pl.ds(row0, ROWS_PER_W)])


@jax.jit
def _sc_call(enc2, steps_w, tgt_w):
    mesh = plsc.VectorSubcoreMesh(core_axis_name="c", subcore_axis_name="s")
    fn = pl.kernel(
        _sc_body,
        out_type=(
            jax.ShapeDtypeStruct((B * T, D), jnp.float32),
            jax.ShapeDtypeStruct((B * T,), jnp.float32),
        ),
        mesh=mesh,
        scratch_types=[
            pltpu.VMEM((L,), jnp.float32),
            pltpu.VMEM((L,), jnp.int32),
            pltpu.VMEM((K,), jnp.int32),
            pltpu.VMEM((K,), jnp.int32),
            pltpu.VMEM((K, D), jnp.float32),
            pltpu.VMEM((K, D), jnp.float32),
            pltpu.VMEM((ROWS_PER_W,), jnp.float32),
            pltpu.SemaphoreType.DMA,
            pltpu.SemaphoreType.DMA,
        ],
    )
    return fn(enc2, steps_w, tgt_w)


def kernel(enc, src_lens, tgt_lens):
    enc2 = enc.reshape(B * S, D)
    steps = (src_lens.astype(jnp.float32) - 1.0) / (
        tgt_lens.astype(jnp.float32) - 1.0)
    steps_w = jnp.broadcast_to(jnp.repeat(steps, NW // B)[:, None], (NW, L))
    tgt_w = jnp.broadcast_to(
        jnp.repeat(tgt_lens.astype(jnp.int32), NW // B)[:, None], (NW, L))
    out, masks = _sc_call(enc2, steps_w, tgt_w)
    return out.reshape(B, T, D), masks.reshape(B, T)
